# baseline (speedup 1.0000x reference)
import functools

import jax
import jax.numpy as jnp
from jax import lax
from jax.experimental import pallas as pl
from jax.experimental.pallas import tpu as pltpu
from jax.experimental.pallas import tpu_sc as plsc

VOCAB = 49408
H = 512
S = 16
BATCH = 4096
N = BATCH * S
L = 16
NC, NS = 2, 16
NW = NC * NS
BPW = N // NW
C = 32
NCHUNK = BPW // C
NBUF = 4
G = 2

_mesh = plsc.VectorSubcoreMesh(core_axis_name="c", subcore_axis_name="s")


@functools.partial(
    pl.kernel,
    out_type=jax.ShapeDtypeStruct((N, H), jnp.float32),
    mesh=_mesh,
    scratch_types=[
        pltpu.VMEM((NCHUNK, C), jnp.int32),
        pltpu.VMEM((S, H), jnp.float32),
        pltpu.VMEM((C, H), jnp.float32),
        pltpu.VMEM((C, H), jnp.float32),
        pltpu.VMEM((C, H), jnp.float32),
        pltpu.VMEM((C, H), jnp.float32),
        pltpu.SemaphoreType.DMA,
        pltpu.SemaphoreType.DMA,
        pltpu.SemaphoreType.DMA,
        pltpu.SemaphoreType.DMA,
        pltpu.SemaphoreType.DMA,
        pltpu.SemaphoreType.DMA,
        pltpu.SemaphoreType.DMA,
        pltpu.SemaphoreType.DMA,
    ],
)
def _emb(ids_hbm, tok_hbm, pos_hbm, out_hbm, idx_v, pos_v,
         b0, b1, b2, b3, g0, g1, g2, g3, s0, s1, s2, s3):
    bufs = (b0, b1, b2, b3)
    gsem = (g0, g1, g2, g3)
    ssem = (s0, s1, s2, s3)
    wid = lax.axis_index("s") * NC + lax.axis_index("c")
    base = wid * BPW
    pltpu.sync_copy(ids_hbm.at[wid], idx_v)
    pltpu.sync_copy(pos_hbm, pos_v)

    def add_pos(rows):
        def jbody(j, c):
            off = j * L
            ps = [pos_v[s, pl.ds(off, L)] for s in range(S)]
            for g in range(C // S):
                for s in range(S):
                    r = g * S + s
                    rows[r, pl.ds(off, L)] = rows[r, pl.ds(off, L)] + ps[s]
            return c
        lax.fori_loop(0, H // L, jbody, 0)

    def fire_gather(k, b):
        return pltpu.async_copy(tok_hbm.at[idx_v.at[k]], bufs[b], gsem[b])

    def wait_gather(k, b):
        pltpu.make_async_copy(tok_hbm.at[idx_v.at[k]], bufs[b], gsem[b]).wait()

    def fire_scatter(k, b):
        return pltpu.async_copy(
            bufs[b], out_hbm.at[pl.ds(base + k * C, C)], ssem[b])

    def wait_scatter(k, b):
        pltpu.make_async_copy(
            bufs[b], out_hbm.at[pl.ds(base + k * C, C)], ssem[b]).wait()

    def step(k, b, swait, gfire):
        wait_gather(k, b)
        bn = (b + G) % NBUF
        if swait:
            wait_scatter(k + G - NBUF, bn)
        if gfire:
            fire_gather(k + G, bn)
        add_pos(bufs[b])
        fire_scatter(k, b)

    for j in range(G):
        fire_gather(j, j)
    for k in range(NBUF - G):
        step(k, k, swait=False, gfire=True)

    n_main = (NCHUNK - NBUF) // NBUF
    base_k = NBUF - G

    def main_wrap(kq, c):
        k0 = base_k + kq * NBUF
        for j in range(NBUF):
            k = k0 + j
            b = (base_k + j) % NBUF
            step(k, b, swait=True, gfire=True)
        return c

    lax.fori_loop(0, n_main, main_wrap, 0)

    for j in range(G):
        k = NCHUNK - G + j
        b = k % NBUF
        step(k, b, swait=True, gfire=False)

    for j in range(NBUF - G):
        k = NCHUNK - (NBUF - G) + j
        wait_scatter(k, k % NBUF)


def kernel(input_ids, token_embedding, position_embedding):
    ids = input_ids.astype(jnp.int32).reshape(NW, NCHUNK, C)
    out = _emb(ids, token_embedding, position_embedding)
    return out.reshape(BATCH, S, H)
